# baseline (speedup 1.0000x reference)
import jax
import jax.numpy as jnp
from jax import lax
from jax.experimental import pallas as pl
from jax.experimental.pallas import tpu as pltpu

B, T, N = 16, 512, 512
FS = 32
NS = N // FS
SP = NS + 1
PAD = -1.0


H = T // 2


def _body(psl_ref, spikes_ref, tm_ref, sm_ref,
          patches_ref, smask_ref, tmask_ref, ss_ref, ts_ref):
    b = pl.program_id(0)
    p = psl_ref[b]
    psl = p % FS
    sidx = p // FS
    do_fix = (psl != 0) & (sidx < NS)

    x = spikes_ref[0]
    lane = lax.broadcasted_iota(jnp.int32, (H, N), 1)
    g = lane // FS
    j = lane - g * FS
    prev = jnp.concatenate(
        [jnp.full((H, FS), PAD, jnp.float32), x[:, : N - FS]], axis=1)
    blended = jnp.where(j < psl, x, prev)
    fixmask = (g == sidx) & do_fix
    patches_ref[0] = jnp.where(fixmask, blended, x).reshape(H * 4, N // 4)

    li = lax.broadcasted_iota(jnp.int32, (H, SP), 1)

    tm = tm_ref[0]
    tmask_ref[0] = jnp.where(li == 0, 1, jnp.broadcast_to(tm, (H, SP)))

    sm = sm_ref[0]
    s_col = jnp.max(sm, axis=1, keepdims=True).astype(jnp.float32)
    s_any = lax.dot_general(
        s_col, jnp.eye(NS, dtype=jnp.float32),
        (((0,), (0,)), ((), ())),
        preferred_element_type=jnp.float32).astype(jnp.int32)
    s_row = jnp.concatenate(
        [jnp.ones((1, 1), jnp.int32), s_any], axis=1)
    smask_ref[0] = jnp.broadcast_to(s_row, (H, SP))

    ss_ref[0] = li
    ts_ref[0] = (lax.broadcasted_iota(jnp.int32, (H, SP), 0)
                 + pl.program_id(1) * H)


def kernel(spikes, pad_space_len, pad_time_len, time_attn_mask,
           space_attn_mask):
    del pad_time_len
    tm3 = time_attn_mask.reshape(B, T, 1)
    sm3 = space_attn_mask.reshape(B, NS, FS)

    grid_spec = pltpu.PrefetchScalarGridSpec(
        num_scalar_prefetch=1,
        grid=(B, 2),
        in_specs=[
            pl.BlockSpec((1, H, N), lambda b, h, psl: (b, h, 0)),
            pl.BlockSpec((1, H, 1), lambda b, h, psl: (b, h, 0)),
            pl.BlockSpec((1, NS, FS), lambda b, h, psl: (b, 0, 0)),
        ],
        out_specs=[
            pl.BlockSpec((1, H * 4, N // 4), lambda b, h, psl: (b, h, 0)),
            pl.BlockSpec((1, H, SP), lambda b, h, psl: (b, h, 0)),
            pl.BlockSpec((1, H, SP), lambda b, h, psl: (b, h, 0)),
            pl.BlockSpec((1, H, SP), lambda b, h, psl: (b, h, 0)),
            pl.BlockSpec((1, H, SP), lambda b, h, psl: (b, h, 0)),
        ],
    )
    patches, smask, tmask, ss, ts = pl.pallas_call(
        _body,
        grid_spec=grid_spec,
        out_shape=[
            jax.ShapeDtypeStruct((B, T * 4, N // 4), jnp.float32),
            jax.ShapeDtypeStruct((B, T, SP), jnp.int32),
            jax.ShapeDtypeStruct((B, T, SP), jnp.int32),
            jax.ShapeDtypeStruct((B, T, SP), jnp.int32),
            jax.ShapeDtypeStruct((B, T, SP), jnp.int32),
        ],
        compiler_params=pltpu.CompilerParams(
            dimension_semantics=("arbitrary", "arbitrary"),
        ),
    )(pad_space_len, spikes, tm3, sm3)

    return (patches.reshape(B, T * NS, FS),
            smask.reshape(B, T * SP),
            tmask.reshape(B, T * SP),
            ss.reshape(B, T * SP),
            ts.reshape(B, T * SP))


# speedup vs baseline: 1.1565x; 1.1565x over previous
import jax
import jax.numpy as jnp
from jax import lax
from jax.experimental import pallas as pl
from jax.experimental.pallas import tpu as pltpu

B, T, N = 16, 512, 512
FS = 32
NS = N // FS
SP = NS + 1
PAD = -1.0


def _body(psl_ref, spikes_ref, tm_ref, sm_ref,
          patches_ref, smask_ref, tmask_ref, ss_ref, ts_ref):
    b = pl.program_id(0)
    p0 = psl_ref[2 * b]
    p1 = psl_ref[2 * b + 1]

    x = spikes_ref[...]
    bi = lax.broadcasted_iota(jnp.int32, (2, T, N), 0)
    p = jnp.where(bi == 0, p0, p1)
    psl = p % FS
    sidx = p // FS
    lane = lax.broadcasted_iota(jnp.int32, (2, T, N), 2)
    g = lane // FS
    j = lane - g * FS
    prev = jnp.concatenate(
        [jnp.full((2, T, FS), PAD, jnp.float32), x[:, :, : N - FS]], axis=2)
    blended = jnp.where(j < psl, x, prev)
    fixmask = (g == sidx) & (psl != 0) & (sidx < NS)
    patches_ref[...] = jnp.where(fixmask, blended, x).reshape(2, T * 4, N // 4)

    li = lax.broadcasted_iota(jnp.int32, (2, T, SP), 2)

    tm = tm_ref[...]
    tmask_ref[...] = jnp.where(li == 0, 1, jnp.broadcast_to(tm, (2, T, SP)))

    sm = sm_ref[...]
    s_col = jnp.max(sm, axis=2, keepdims=True).astype(jnp.float32)
    s_col2 = jnp.concatenate([s_col[0], s_col[1]], axis=1)
    s_any2 = lax.dot_general(
        s_col2, jnp.eye(NS, dtype=jnp.float32),
        (((0,), (0,)), ((), ())),
        preferred_element_type=jnp.float32).astype(jnp.int32)
    s_row = jnp.concatenate(
        [jnp.ones((2, 1), jnp.int32), s_any2], axis=1)[:, None, :]
    smask_ref[...] = jnp.broadcast_to(s_row, (2, T, SP))

    ss_ref[...] = li
    ts_ref[...] = lax.broadcasted_iota(jnp.int32, (2, T, SP), 1)


def kernel(spikes, pad_space_len, pad_time_len, time_attn_mask,
           space_attn_mask):
    del pad_time_len
    tm3 = time_attn_mask.reshape(B, T, 1)
    sm3 = space_attn_mask.reshape(B, NS, FS)

    grid_spec = pltpu.PrefetchScalarGridSpec(
        num_scalar_prefetch=1,
        grid=(B // 2,),
        in_specs=[
            pl.BlockSpec((2, T, N), lambda b, psl: (b, 0, 0)),
            pl.BlockSpec((2, T, 1), lambda b, psl: (b, 0, 0)),
            pl.BlockSpec((2, NS, FS), lambda b, psl: (b, 0, 0)),
        ],
        out_specs=[
            pl.BlockSpec((2, T * 4, N // 4), lambda b, psl: (b, 0, 0)),
            pl.BlockSpec((2, T, SP), lambda b, psl: (b, 0, 0)),
            pl.BlockSpec((2, T, SP), lambda b, psl: (b, 0, 0)),
            pl.BlockSpec((2, T, SP), lambda b, psl: (b, 0, 0)),
            pl.BlockSpec((2, T, SP), lambda b, psl: (b, 0, 0)),
        ],
    )
    patches, smask, tmask, ss, ts = pl.pallas_call(
        _body,
        grid_spec=grid_spec,
        out_shape=[
            jax.ShapeDtypeStruct((B, T * 4, N // 4), jnp.float32),
            jax.ShapeDtypeStruct((B, T, SP), jnp.int32),
            jax.ShapeDtypeStruct((B, T, SP), jnp.int32),
            jax.ShapeDtypeStruct((B, T, SP), jnp.int32),
            jax.ShapeDtypeStruct((B, T, SP), jnp.int32),
        ],
        compiler_params=pltpu.CompilerParams(
            dimension_semantics=("arbitrary",),
        ),
    )(pad_space_len, spikes, tm3, sm3)

    return (patches.reshape(B, T * NS, FS),
            smask.reshape(B, T * SP),
            tmask.reshape(B, T * SP),
            ss.reshape(B, T * SP),
            ts.reshape(B, T * SP))


# speedup vs baseline: 1.1785x; 1.0190x over previous
import jax
import jax.numpy as jnp
from jax import lax
from jax.experimental import pallas as pl
from jax.experimental.pallas import tpu as pltpu

B, T, N = 16, 512, 512
FS = 32
NS = N // FS
SP = NS + 1
PAD = -1.0
G = 4


def _body(psl_ref, spikes_ref, tm_ref, sm_ref,
          patches_ref, smask_ref, tmask_ref, ss_ref, ts_ref):
    b = pl.program_id(0)
    p0 = psl_ref[4 * b]
    p1 = psl_ref[4 * b + 1]
    p2 = psl_ref[4 * b + 2]
    p3 = psl_ref[4 * b + 3]

    x = spikes_ref[...]
    bi = lax.broadcasted_iota(jnp.int32, (G, T, N), 0)
    p = jnp.where(bi == 0, p0,
                  jnp.where(bi == 1, p1, jnp.where(bi == 2, p2, p3)))
    psl = p % FS
    sidx = p // FS
    lane = lax.broadcasted_iota(jnp.int32, (G, T, N), 2)
    g = lane // FS
    j = lane - g * FS
    prev = jnp.concatenate(
        [jnp.full((G, T, FS), PAD, jnp.float32), x[:, :, : N - FS]], axis=2)
    blended = jnp.where(j < psl, x, prev)
    fixmask = (g == sidx) & (psl != 0) & (sidx < NS)
    patches_ref[...] = jnp.where(fixmask, blended, x).reshape(G, T * 4, N // 4)

    li = lax.broadcasted_iota(jnp.int32, (G, T, SP), 2)

    tm = tm_ref[...]
    tmask_ref[...] = jnp.where(li == 0, 1, jnp.broadcast_to(tm, (G, T, SP)))

    sm = sm_ref[...]
    s_col = jnp.max(sm, axis=2, keepdims=True).astype(jnp.float32)
    s_col2 = jnp.concatenate([s_col[i] for i in range(G)], axis=1)
    s_any2 = lax.dot_general(
        s_col2, jnp.eye(NS, dtype=jnp.float32),
        (((0,), (0,)), ((), ())),
        preferred_element_type=jnp.float32).astype(jnp.int32)
    s_row = jnp.concatenate(
        [jnp.ones((G, 1), jnp.int32), s_any2], axis=1)[:, None, :]
    smask_ref[...] = jnp.broadcast_to(s_row, (G, T, SP))

    ss_ref[...] = li
    ts_ref[...] = lax.broadcasted_iota(jnp.int32, (G, T, SP), 1)


def kernel(spikes, pad_space_len, pad_time_len, time_attn_mask,
           space_attn_mask):
    del pad_time_len
    tm3 = time_attn_mask.reshape(B, T, 1)
    sm3 = space_attn_mask.reshape(B, NS, FS)

    grid_spec = pltpu.PrefetchScalarGridSpec(
        num_scalar_prefetch=1,
        grid=(B // G,),
        in_specs=[
            pl.BlockSpec((G, T, N), lambda b, psl: (b, 0, 0)),
            pl.BlockSpec((G, T, 1), lambda b, psl: (b, 0, 0)),
            pl.BlockSpec((G, NS, FS), lambda b, psl: (b, 0, 0)),
        ],
        out_specs=[
            pl.BlockSpec((G, T * 4, N // 4), lambda b, psl: (b, 0, 0)),
            pl.BlockSpec((G, T, SP), lambda b, psl: (b, 0, 0)),
            pl.BlockSpec((G, T, SP), lambda b, psl: (b, 0, 0)),
            pl.BlockSpec((G, T, SP), lambda b, psl: (b, 0, 0)),
            pl.BlockSpec((G, T, SP), lambda b, psl: (b, 0, 0)),
        ],
    )
    patches, smask, tmask, ss, ts = pl.pallas_call(
        _body,
        grid_spec=grid_spec,
        out_shape=[
            jax.ShapeDtypeStruct((B, T * 4, N // 4), jnp.float32),
            jax.ShapeDtypeStruct((B, T, SP), jnp.int32),
            jax.ShapeDtypeStruct((B, T, SP), jnp.int32),
            jax.ShapeDtypeStruct((B, T, SP), jnp.int32),
            jax.ShapeDtypeStruct((B, T, SP), jnp.int32),
        ],
        compiler_params=pltpu.CompilerParams(
            dimension_semantics=("arbitrary",),
        ),
    )(pad_space_len, spikes, tm3, sm3)

    return (patches.reshape(B, T * NS, FS),
            smask.reshape(B, T * SP),
            tmask.reshape(B, T * SP),
            ss.reshape(B, T * SP),
            ts.reshape(B, T * SP))
